# baseline (speedup 1.0000x reference)
import functools

import jax
import jax.numpy as jnp
from jax import lax
from jax.experimental import pallas as pl
from jax.experimental.pallas import tpu as pltpu
from jax.experimental.pallas import tpu_sc as plsc

N = 10000
E = 320000
D = 128
EPS = 1e-5

NC = 2
NS = 16
NW = NC * NS
L = 16

N_PAD = 10240
RPT = N_PAD // NS
EPT = 10240
E_PAD = NW * EPT
C = 128
NCHUNK = EPT // C

_mesh = plsc.VectorSubcoreMesh(core_axis_name="c", subcore_axis_name="s")


@functools.partial(
    pl.kernel,
    out_type=jax.ShapeDtypeStruct((NW, N_PAD), jnp.float32),
    mesh=_mesh,
    scratch_types=[
        pltpu.VMEM((N_PAD,), jnp.float32),
        pltpu.VMEM((EPT,), jnp.int32),
    ],
    compiler_params=pltpu.CompilerParams(needs_layout_passes=False),
)
def _sc_degree(dst_hbm, out_hbm, hist, idxbuf):
    cid = lax.axis_index("c")
    sid = lax.axis_index("s")
    wid = sid * NC + cid

    zeros = jnp.zeros((L,), jnp.float32)

    @pl.loop(0, N_PAD // L)
    def _zero(k):
        hist[pl.ds(k * L, L)] = zeros

    pltpu.sync_copy(dst_hbm.at[pl.ds(wid * EPT, EPT)], idxbuf)

    ones = jnp.ones((L,), jnp.float32)

    @pl.loop(0, EPT // L)
    def _count(k):
        idx = idxbuf[pl.ds(k * L, L)]
        plsc.addupdate_scatter(hist, [idx], ones)

    pltpu.sync_copy(hist, out_hbm.at[wid])


NSLOT = 2


@functools.partial(
    pl.kernel,
    out_type=jax.ShapeDtypeStruct((NC, N_PAD, D), jnp.float32),
    mesh=_mesh,
    scratch_types=[
        pltpu.VMEM_SHARED((N_PAD, D), jnp.float32),
        pltpu.VMEM((NSLOT, 1, C), jnp.int32),
        pltpu.VMEM((NSLOT, 1, C), jnp.int32),
        pltpu.VMEM((NSLOT, C, D), jnp.float32),
        pltpu.VMEM((L, D), jnp.float32),
        pltpu.SemaphoreType.DMA((NSLOT,)),
        pltpu.SemaphoreType.DMA((NSLOT,)),
    ],
)
def _sc_msgpass(y_hbm, src_hbm, dst_hbm, out_hbm, acc, sidx, didx, rows, zbuf,
                gsem, isem):
    cid = lax.axis_index("c")
    sid = lax.axis_index("s")
    wid = sid * NC + cid
    eb = wid * NCHUNK

    def _idx_start(g, slot):
        pltpu.async_copy(src_hbm.at[pl.ds(eb + g, 1)], sidx.at[slot],
                         isem.at[slot])
        pltpu.async_copy(dst_hbm.at[pl.ds(eb + g, 1)], didx.at[slot],
                         isem.at[slot])

    def _idx_wait(slot):
        pltpu.make_async_copy(src_hbm.at[pl.ds(eb, 1)], sidx.at[slot],
                              isem.at[slot]).wait()
        pltpu.make_async_copy(dst_hbm.at[pl.ds(eb, 1)], didx.at[slot],
                              isem.at[slot]).wait()

    def _gather_start(slot):
        pltpu.async_copy(y_hbm.at[sidx.at[slot, 0]], rows.at[slot],
                         gsem.at[slot])

    def _gather_wait(slot):
        pltpu.make_async_copy(y_hbm.at[sidx.at[slot, 0]], rows.at[slot],
                              gsem.at[slot]).wait()

    _idx_start(0, 0)
    _idx_start(1, 1)

    zeros = jnp.zeros((L,), jnp.float32)
    for r in range(L):
        for j in range(D // L):
            zbuf[r, pl.ds(j * L, L)] = zeros

    @pl.loop(0, RPT // L)
    def _zero(k):
        pltpu.sync_copy(zbuf, acc.at[pl.ds(sid * RPT + k * L, L)])

    plsc.subcore_barrier()

    _idx_wait(0)
    _gather_start(0)

    @pl.loop(0, NCHUNK // NSLOT)
    def _super(i):
        g0 = i * NSLOT
        for slot in range(NSLOT):
            g = g0 + slot
            other = 1 - slot

            @pl.when(g + 1 < NCHUNK)
            def _():
                _idx_wait(other)
                _gather_start(other)

            _gather_wait(slot)
            pltpu.sync_copy(rows.at[slot], acc.at[didx.at[slot, 0]], add=True)

            @pl.when(g + NSLOT < NCHUNK)
            def _():
                _idx_start(g + NSLOT, slot)

    plsc.subcore_barrier()

    pltpu.sync_copy(acc.at[pl.ds(sid * RPT, RPT)],
                    out_hbm.at[cid, pl.ds(sid * RPT, RPT)])


def _tc1_body(x_ref, g_ref, b_ref, w_ref, degp_ref, y_ref, dinv_ref):
    x = x_ref[...]
    m = jnp.mean(x, axis=0)
    v = jnp.mean(jnp.square(x - m), axis=0)
    h = (x - m) * lax.rsqrt(v + EPS) * g_ref[...] + b_ref[...]
    deg = 1.0 + jnp.sum(degp_ref[...][:, :N], axis=0)
    dinv = lax.rsqrt(deg)[:, None]
    y_ref[...] = jnp.dot(h, w_ref[...],
                         preferred_element_type=jnp.float32) * dinv
    dinv_ref[...] = dinv


def _tc2_body(a0_ref, a1_ref, y1_ref, dinv_ref, b1_ref, g_ref, b_ref, w_ref,
              y2_ref):
    dinv = dinv_ref[...]
    out1 = (a0_ref[...] + a1_ref[...] + y1_ref[...]) * dinv + b1_ref[...]
    m = jnp.mean(out1, axis=0)
    v = jnp.mean(jnp.square(out1 - m), axis=0)
    h = (out1 - m) * lax.rsqrt(v + EPS) * g_ref[...] + b_ref[...]
    y2_ref[...] = jnp.dot(h, w_ref[...],
                          preferred_element_type=jnp.float32) * dinv


def _tc3_body(a0_ref, a1_ref, y2_ref, dinv_ref, b2_ref, out_ref):
    out = (a0_ref[...] + a1_ref[...] + y2_ref[...]) * dinv_ref[...] \
        + b2_ref[...]
    out_ref[...] = jnp.maximum(out, 0.0)


_tc1 = pl.pallas_call(
    _tc1_body,
    out_shape=(jax.ShapeDtypeStruct((N, D), jnp.float32),
               jax.ShapeDtypeStruct((N, 1), jnp.float32)),
)
_tc2 = pl.pallas_call(
    _tc2_body,
    out_shape=jax.ShapeDtypeStruct((N, D), jnp.float32),
)
_tc3 = pl.pallas_call(
    _tc3_body,
    out_shape=jax.ShapeDtypeStruct((N, D), jnp.float32),
)


@jax.jit
def kernel(x, edge_index, bn_in_g, bn_in_b, W1, b1, bn_h_g, bn_h_b, W2, b2):
    pad = E_PAD - E
    src = jnp.concatenate([edge_index[0], jnp.zeros((pad,), jnp.int32)])
    trash = N + jnp.arange(pad, dtype=jnp.int32) % (N_PAD - N)
    dst = jnp.concatenate([edge_index[1], trash])
    src2d = src.reshape(NW * NCHUNK, C)
    dst2d = dst.reshape(NW * NCHUNK, C)

    degp = _sc_degree(dst)
    y1, dinv = _tc1(x, bn_in_g, bn_in_b, W1, degp)

    acc1 = _sc_msgpass(y1, src2d, dst2d)
    y2 = _tc2(acc1[0, :N], acc1[1, :N], y1, dinv, b1, bn_h_g, bn_h_b, W2)

    acc2 = _sc_msgpass(y2, src2d, dst2d)
    return _tc3(acc2[0, :N], acc2[1, :N], y2, dinv, b2)


# speedup vs baseline: 1.2418x; 1.2418x over previous
import functools

import jax
import jax.numpy as jnp
from jax import lax
from jax.experimental import pallas as pl
from jax.experimental.pallas import tpu as pltpu
from jax.experimental.pallas import tpu_sc as plsc

N = 10000
E = 320000
D = 128
EPS = 1e-5

NC = 2
NS = 16
NW = NC * NS
L = 16

N_PAD = 10240
RPT = N_PAD // NS
EPT = 10240
E_PAD = NW * EPT
C = 128
NCHUNK = EPT // C

_mesh = plsc.VectorSubcoreMesh(core_axis_name="c", subcore_axis_name="s")


@functools.partial(
    pl.kernel,
    out_type=jax.ShapeDtypeStruct((NW, N_PAD), jnp.float32),
    mesh=_mesh,
    scratch_types=[
        pltpu.VMEM((N_PAD,), jnp.float32),
        pltpu.VMEM((EPT,), jnp.int32),
    ],
    compiler_params=pltpu.CompilerParams(needs_layout_passes=False),
)
def _sc_degree(dst_hbm, out_hbm, hist, idxbuf):
    cid = lax.axis_index("c")
    sid = lax.axis_index("s")
    wid = sid * NC + cid

    zeros = jnp.zeros((L,), jnp.float32)

    @pl.loop(0, N_PAD // L)
    def _zero(k):
        hist[pl.ds(k * L, L)] = zeros

    pltpu.sync_copy(dst_hbm.at[pl.ds(wid * EPT, EPT)], idxbuf)

    ones = jnp.ones((L,), jnp.float32)

    @pl.loop(0, EPT // L)
    def _count(k):
        idx = idxbuf[pl.ds(k * L, L)]
        plsc.addupdate_scatter(hist, [idx], ones)

    pltpu.sync_copy(hist, out_hbm.at[wid])


NSLOT = 2
SPT = 160
K0 = 126
K1 = SPT - K0


@functools.partial(
    pl.kernel,
    out_type=jax.ShapeDtypeStruct((NC, N_PAD, D), jnp.float32),
    mesh=_mesh,
    scratch_types=[
        pltpu.VMEM_SHARED((N_PAD, D), jnp.float32),
        pltpu.VMEM((NSLOT, 1, C), jnp.int32),
        pltpu.VMEM((NSLOT, 1, C), jnp.int32),
        pltpu.VMEM((NSLOT, C, D), jnp.float32),
        pltpu.VMEM((L, D), jnp.float32),
        pltpu.SemaphoreType.DMA((NSLOT,)),
        pltpu.SemaphoreType.DMA((NSLOT,)),
    ],
)
def _sc_msgpass(y_hbm, src_hbm, dst_hbm, out_hbm, acc, sidx, didx, rows, zbuf,
                gsem, isem):
    cid = lax.axis_index("c")
    sid = lax.axis_index("s")

    def _idx_start(eb, g, slot):
        pltpu.async_copy(src_hbm.at[pl.ds(eb + g, 1)], sidx.at[slot],
                         isem.at[slot])
        pltpu.async_copy(dst_hbm.at[pl.ds(eb + g, 1)], didx.at[slot],
                         isem.at[slot])

    def _idx_wait(eb, slot):
        pltpu.make_async_copy(src_hbm.at[pl.ds(eb, 1)], sidx.at[slot],
                              isem.at[slot]).wait()
        pltpu.make_async_copy(dst_hbm.at[pl.ds(eb, 1)], didx.at[slot],
                              isem.at[slot]).wait()

    def _gather_start(slot):
        pltpu.async_copy(y_hbm.at[sidx.at[slot, 0]], rows.at[slot],
                         gsem.at[slot])

    def _gather_wait(slot):
        pltpu.make_async_copy(y_hbm.at[sidx.at[slot, 0]], rows.at[slot],
                              gsem.at[slot]).wait()

    def _pipeline(eb, nchunk):
        _idx_start(eb, 0, 0)
        _idx_start(eb, 1, 1)
        _idx_wait(eb, 0)
        _gather_start(0)

        @pl.loop(0, nchunk // NSLOT)
        def _super(i):
            g0 = i * NSLOT
            for slot in range(NSLOT):
                g = g0 + slot
                other = 1 - slot

                @pl.when(g + 1 < nchunk)
                def _():
                    _idx_wait(eb, other)
                    _gather_start(other)

                _gather_wait(slot)
                pltpu.sync_copy(rows.at[slot], acc.at[didx.at[slot, 0]],
                                add=True)

                @pl.when(g + NSLOT < nchunk)
                def _():
                    _idx_start(eb, g + NSLOT, slot)

    zeros = jnp.zeros((L,), jnp.float32)
    for r in range(L):
        for j in range(D // L):
            zbuf[r, pl.ds(j * L, L)] = zeros

    @pl.loop(0, RPT // L)
    def _zero(k):
        pltpu.sync_copy(zbuf, acc.at[pl.ds(sid * RPT + k * L, L)])

    plsc.subcore_barrier()

    @pl.when(cid == 0)
    def _():
        _pipeline(sid * SPT, K0)

    @pl.when(cid == 1)
    def _():
        _pipeline(sid * SPT + K0, K1)

    plsc.subcore_barrier()

    pltpu.sync_copy(acc.at[pl.ds(sid * RPT, RPT)],
                    out_hbm.at[cid, pl.ds(sid * RPT, RPT)])


def _tc1_body(x_ref, g_ref, b_ref, w_ref, degp_ref, y_ref, dinv_ref):
    x = x_ref[...]
    m = jnp.mean(x, axis=0)
    v = jnp.mean(jnp.square(x - m), axis=0)
    h = (x - m) * lax.rsqrt(v + EPS) * g_ref[...] + b_ref[...]
    deg = 1.0 + jnp.sum(degp_ref[...][:, :N], axis=0)
    dinv = lax.rsqrt(deg)[:, None]
    y_ref[...] = jnp.dot(h, w_ref[...],
                         preferred_element_type=jnp.float32) * dinv
    dinv_ref[...] = dinv


def _tc2_body(a0_ref, a1_ref, y1_ref, dinv_ref, b1_ref, g_ref, b_ref, w_ref,
              y2_ref):
    dinv = dinv_ref[...]
    out1 = (a0_ref[...] + a1_ref[...] + y1_ref[...]) * dinv + b1_ref[...]
    m = jnp.mean(out1, axis=0)
    v = jnp.mean(jnp.square(out1 - m), axis=0)
    h = (out1 - m) * lax.rsqrt(v + EPS) * g_ref[...] + b_ref[...]
    y2_ref[...] = jnp.dot(h, w_ref[...],
                          preferred_element_type=jnp.float32) * dinv


def _tc3_body(a0_ref, a1_ref, y2_ref, dinv_ref, b2_ref, out_ref):
    out = (a0_ref[...] + a1_ref[...] + y2_ref[...]) * dinv_ref[...] \
        + b2_ref[...]
    out_ref[...] = jnp.maximum(out, 0.0)


_tc1 = pl.pallas_call(
    _tc1_body,
    out_shape=(jax.ShapeDtypeStruct((N, D), jnp.float32),
               jax.ShapeDtypeStruct((N, 1), jnp.float32)),
)
_tc2 = pl.pallas_call(
    _tc2_body,
    out_shape=jax.ShapeDtypeStruct((N, D), jnp.float32),
)
_tc3 = pl.pallas_call(
    _tc3_body,
    out_shape=jax.ShapeDtypeStruct((N, D), jnp.float32),
)


@jax.jit
def kernel(x, edge_index, bn_in_g, bn_in_b, W1, b1, bn_h_g, bn_h_b, W2, b2):
    pad = E_PAD - E
    src = jnp.concatenate([edge_index[0], jnp.zeros((pad,), jnp.int32)])
    dst = jnp.concatenate([edge_index[1], jnp.full((pad,), N, jnp.int32)])
    src2d = src.reshape(NW * NCHUNK, C)
    dst2d = dst.reshape(NW * NCHUNK, C)

    degp = _sc_degree(dst)
    y1, dinv = _tc1(x, bn_in_g, bn_in_b, W1, degp)

    acc1 = _sc_msgpass(y1, src2d, dst2d)
    y2 = _tc2(acc1[0, :N], acc1[1, :N], y1, dinv, b1, bn_h_g, bn_h_b, W2)

    acc2 = _sc_msgpass(y2, src2d, dst2d)
    return _tc3(acc2[0, :N], acc2[1, :N], y2, dinv, b2)
